# Initial kernel scaffold; baseline (speedup 1.0000x reference)
import jax
import jax.numpy as jnp
from jax.experimental import pallas as pl


def kernel(indices, word_table, pos_table):
    raise NotImplementedError("write your pallas kernel here")


# speedup vs baseline: 1.0231x; 1.0231x over previous
import jax
import jax.numpy as jnp
from jax import lax
from jax.experimental import pallas as pl
from jax.experimental.pallas import tpu as pltpu
from jax.experimental.pallas import tpu_sc as plsc

_B, _S, _H = 4, 2048, 768
_NC, _NS = 2, 16
_NW = _NC * _NS
_SPW = _S // _NW
_R = 32
_NCHUNK = _SPW // _R
_LANES = 16


def _body(idx_hbm, wt_hbm, pt_hbm, out_hbm, idx_v, pos_v, w_v, gsem):
    cid = lax.axis_index("c")
    sid = lax.axis_index("s")
    wid = sid * _NC + cid
    s0 = wid * _SPW
    pltpu.sync_copy(pt_hbm.at[pl.ds(s0, _SPW)], pos_v)
    for b in range(_B):
        for c in range(_NCHUNK):
            r0 = s0 + c * _R
            pltpu.sync_copy(idx_hbm.at[b, pl.ds(r0, _R)], idx_v)
            pltpu.async_copy(wt_hbm.at[idx_v], w_v, gsem).wait()

            def add_row(r, carry, c=c):
                for j in range(_H // _LANES):
                    x = pos_v[c * _R + r, pl.ds(j * _LANES, _LANES)]
                    plsc.addupdate(w_v.at[r, pl.ds(j * _LANES, _LANES)], x)
                return carry

            lax.fori_loop(0, _R, add_row, 0)
            pltpu.sync_copy(w_v, out_hbm.at[b, pl.ds(r0, _R)])


def kernel(indices, word_table, pos_table):
    idx = indices.astype(jnp.int32)
    mesh = plsc.VectorSubcoreMesh(
        core_axis_name="c", subcore_axis_name="s",
        num_cores=_NC, num_subcores=_NS)
    k = pl.kernel(
        _body,
        out_type=jax.ShapeDtypeStruct((_B, _S, _H), jnp.float32),
        mesh=mesh,
        scratch_types=[
            pltpu.VMEM((_R,), jnp.int32),
            pltpu.VMEM((_SPW, _H), jnp.float32),
            pltpu.VMEM((_R, _H), jnp.float32),
            pltpu.SemaphoreType.DMA,
        ],
    )
    return k(idx, word_table, pos_table)


# speedup vs baseline: 1.0496x; 1.0259x over previous
import jax
import jax.numpy as jnp
from jax import lax
from jax.experimental import pallas as pl
from jax.experimental.pallas import tpu as pltpu
from jax.experimental.pallas import tpu_sc as plsc

_B, _S, _H = 4, 2048, 768
_NC, _NS = 2, 16
_NW = _NC * _NS
_SPW = _S // _NW
_R = 32
_NCHUNK = _SPW // _R
_NCH = _B * _NCHUNK
_LANES = 16


def _body(idx_hbm, wt_hbm, pt_hbm, out_hbm, idx_v, pos_v, w_v,
          psem, gsem0, gsem1, osem0, osem1):
    cid = lax.axis_index("c")
    sid = lax.axis_index("s")
    wid = sid * _NC + cid
    s0 = wid * _SPW

    gsems = [gsem0, gsem1]
    osems = [osem0, osem1]

    ph = pltpu.async_copy(pt_hbm.at[pl.ds(s0, _SPW)], pos_v, psem)
    for b in range(_B):
        pltpu.sync_copy(idx_hbm.at[b, pl.ds(s0, _SPW)], idx_v.at[b])

    gh = [None, None]
    oh = [None, None]

    def start_gather(k):
        p = k & 1
        b, c = divmod(k, _NCHUNK)
        gh[p] = pltpu.async_copy(
            wt_hbm.at[idx_v.at[b, pl.ds(c * _R, _R)]], w_v.at[p], gsems[p])

    start_gather(0)
    ph.wait()
    for k in range(_NCH):
        p = k & 1
        b, c = divmod(k, _NCHUNK)
        if k + 1 < _NCH:
            if oh[1 - p] is not None:
                oh[1 - p].wait()
                oh[1 - p] = None
            start_gather(k + 1)
        gh[p].wait()

        def add_row(r, carry, c=c, p=p):
            for j in range(_H // _LANES):
                x = pos_v[c * _R + r, pl.ds(j * _LANES, _LANES)]
                plsc.addupdate(w_v.at[p, r, pl.ds(j * _LANES, _LANES)], x)
            return carry

        lax.fori_loop(0, _R, add_row, 0)
        oh[p] = pltpu.async_copy(
            w_v.at[p], out_hbm.at[b, pl.ds(s0 + c * _R, _R)], osems[p])
    oh[0].wait()
    oh[1].wait()


def kernel(indices, word_table, pos_table):
    idx = indices.astype(jnp.int32)
    mesh = plsc.VectorSubcoreMesh(
        core_axis_name="c", subcore_axis_name="s",
        num_cores=_NC, num_subcores=_NS)
    k = pl.kernel(
        _body,
        out_type=jax.ShapeDtypeStruct((_B, _S, _H), jnp.float32),
        mesh=mesh,
        scratch_types=[
            pltpu.VMEM((_B, _SPW), jnp.int32),
            pltpu.VMEM((_SPW, _H), jnp.float32),
            pltpu.VMEM((2, _R, _H), jnp.float32),
            pltpu.SemaphoreType.DMA,
            pltpu.SemaphoreType.DMA,
            pltpu.SemaphoreType.DMA,
            pltpu.SemaphoreType.DMA,
            pltpu.SemaphoreType.DMA,
        ],
    )
    return k(idx, word_table, pos_table)


# speedup vs baseline: 1.3339x; 1.2708x over previous
import jax
import jax.numpy as jnp
from jax import lax
from jax.experimental import pallas as pl
from jax.experimental.pallas import tpu as pltpu
from jax.experimental.pallas import tpu_sc as plsc

_B, _S, _H = 4, 2048, 768
_NC, _NS = 2, 16
_NW = _NC * _NS
_SPW = _S // _NW
_R = 16
_NCHUNK = _SPW // _R
_LANES = 16


def _body(idx_hbm, wt_hbm, pt_hbm, out_hbm, idx_v, pos_v, w_v,
          psem0, psem1, gsem0, gsem1, osem0, osem1):
    cid = lax.axis_index("c")
    sid = lax.axis_index("s")
    wid = sid * _NC + cid
    s0 = wid * _SPW

    psems = [psem0, psem1]
    gsems = [gsem0, gsem1]
    osems = [osem0, osem1]

    for b in range(_B):
        pltpu.sync_copy(idx_hbm.at[b, pl.ds(s0, _SPW)], idx_v.at[b])

    ph = [None, None]
    gh = [[None] * _B, [None] * _B]
    oh = [[None] * _B, [None] * _B]

    def start_chunk(c):
        p = c & 1
        ph[p] = pltpu.async_copy(
            pt_hbm.at[pl.ds(s0 + c * _R, _R)], pos_v.at[p], psems[p])
        for b in range(_B):
            gh[p][b] = pltpu.async_copy(
                wt_hbm.at[idx_v.at[b, pl.ds(c * _R, _R)]], w_v.at[p, b],
                gsems[p])

    start_chunk(0)
    for c in range(_NCHUNK):
        p = c & 1
        if c + 1 < _NCHUNK:
            if oh[1 - p][0] is not None:
                for b in range(_B):
                    oh[1 - p][b].wait()
                    oh[1 - p][b] = None
            start_chunk(c + 1)
        ph[p].wait()
        for b in range(_B):
            gh[p][b].wait()

        def add_row(r, carry, p=p):
            for j in range(_H // _LANES):
                x = pos_v[p, r, pl.ds(j * _LANES, _LANES)]
                for b in range(_B):
                    plsc.addupdate(
                        w_v.at[p, b, r, pl.ds(j * _LANES, _LANES)], x)
            return carry

        lax.fori_loop(0, _R, add_row, 0)
        for b in range(_B):
            oh[p][b] = pltpu.async_copy(
                w_v.at[p, b], out_hbm.at[b, pl.ds(s0 + c * _R, _R)],
                osems[p])
    for p in range(2):
        for b in range(_B):
            oh[p][b].wait()


def kernel(indices, word_table, pos_table):
    idx = indices.astype(jnp.int32)
    mesh = plsc.VectorSubcoreMesh(
        core_axis_name="c", subcore_axis_name="s",
        num_cores=_NC, num_subcores=_NS)
    k = pl.kernel(
        _body,
        out_type=jax.ShapeDtypeStruct((_B, _S, _H), jnp.float32),
        mesh=mesh,
        scratch_types=[
            pltpu.VMEM((_B, _SPW), jnp.int32),
            pltpu.VMEM((2, _R, _H), jnp.float32),
            pltpu.VMEM((2, _B, _R, _H), jnp.float32),
            pltpu.SemaphoreType.DMA,
            pltpu.SemaphoreType.DMA,
            pltpu.SemaphoreType.DMA,
            pltpu.SemaphoreType.DMA,
            pltpu.SemaphoreType.DMA,
            pltpu.SemaphoreType.DMA,
        ],
    )
    return k(idx, word_table, pos_table)


# speedup vs baseline: 1.3594x; 1.0191x over previous
import jax
import jax.numpy as jnp
from jax import lax
from jax.experimental import pallas as pl
from jax.experimental.pallas import tpu as pltpu
from jax.experimental.pallas import tpu_sc as plsc

_B, _S, _H = 4, 2048, 768
_NC, _NS = 2, 16
_NW = _NC * _NS
_SPW = _S // _NW
_R = 8
_NCHUNK = _SPW // _R
_NBUF = 3
_LANES = 16


def _body(idx_hbm, wt_hbm, pt_hbm, out_hbm, idx_v, pos_v, w_v,
          psem, gsem0, gsem1, gsem2, osem0, osem1, osem2):
    cid = lax.axis_index("c")
    sid = lax.axis_index("s")
    wid = sid * _NC + cid
    s0 = wid * _SPW

    gsems = [gsem0, gsem1, gsem2]
    osems = [osem0, osem1, osem2]

    ph = pltpu.async_copy(pt_hbm.at[pl.ds(s0, _SPW)], pos_v, psem)
    for b in range(_B):
        pltpu.sync_copy(idx_hbm.at[b, pl.ds(s0, _SPW)], idx_v.at[b])

    gh = [[None] * _B for _ in range(_NBUF)]
    oh = [[None] * _B for _ in range(_NBUF)]

    def start_gathers(c):
        p = c % _NBUF
        for b in range(_B):
            gh[p][b] = pltpu.async_copy(
                wt_hbm.at[idx_v.at[b, pl.ds(c * _R, _R)]], w_v.at[p, b],
                gsems[p])

    for c in range(_NBUF - 1):
        start_gathers(c)
    ph.wait()
    for c in range(_NCHUNK):
        p = c % _NBUF
        cn = c + _NBUF - 1
        if cn < _NCHUNK:
            pn = cn % _NBUF
            if oh[pn][0] is not None:
                for b in range(_B):
                    oh[pn][b].wait()
                    oh[pn][b] = None
            start_gathers(cn)
        for b in range(_B):
            gh[p][b].wait()

        def add_row(r, carry, c=c, p=p):
            for j in range(_H // _LANES):
                x = pos_v[c * _R + r, pl.ds(j * _LANES, _LANES)]
                for b in range(_B):
                    plsc.addupdate(
                        w_v.at[p, b, r, pl.ds(j * _LANES, _LANES)], x)
            return carry

        lax.fori_loop(0, _R, add_row, 0)
        for b in range(_B):
            oh[p][b] = pltpu.async_copy(
                w_v.at[p, b], out_hbm.at[b, pl.ds(s0 + c * _R, _R)],
                osems[p])
    for c in range(_NCHUNK - _NBUF, _NCHUNK):
        for b in range(_B):
            oh[c % _NBUF][b].wait()


def kernel(indices, word_table, pos_table):
    idx = indices.astype(jnp.int32)
    mesh = plsc.VectorSubcoreMesh(
        core_axis_name="c", subcore_axis_name="s",
        num_cores=_NC, num_subcores=_NS)
    k = pl.kernel(
        _body,
        out_type=jax.ShapeDtypeStruct((_B, _S, _H), jnp.float32),
        mesh=mesh,
        scratch_types=[
            pltpu.VMEM((_B, _SPW), jnp.int32),
            pltpu.VMEM((_SPW, _H), jnp.float32),
            pltpu.VMEM((_NBUF, _B, _R, _H), jnp.float32),
            pltpu.SemaphoreType.DMA,
            pltpu.SemaphoreType.DMA,
            pltpu.SemaphoreType.DMA,
            pltpu.SemaphoreType.DMA,
            pltpu.SemaphoreType.DMA,
            pltpu.SemaphoreType.DMA,
            pltpu.SemaphoreType.DMA,
        ],
    )
    return k(idx, word_table, pos_table)
